# baseline (speedup 1.0000x reference)
import functools

import jax
import jax.numpy as jnp
from jax import lax
from jax.experimental import pallas as pl
from jax.experimental.pallas import tpu as pltpu
from jax.experimental.pallas import tpu_sc as plsc

_B = 16384
_C = 1000
_NC, _NS = 2, 16
_NW = _NC * _NS
_PER = _B // _NW
_RJ = 4
_RL = _PER // _RJ
_LANES = 16


def _sc_loss(preds_flat, targets, margin):
    mesh = plsc.VectorSubcoreMesh(core_axis_name="c", subcore_axis_name="s")

    @functools.partial(
        pl.kernel,
        mesh=mesh,
        out_type=jax.ShapeDtypeStruct((_NC, _LANES), jnp.float32),
        compiler_params=pltpu.CompilerParams(needs_layout_passes=False),
        scratch_types=[
            pltpu.VMEM((_PER,), jnp.int32),
            pltpu.VMEM((_PER,), jnp.float32),
            pltpu.VMEM((_PER,), jnp.float32),
            pltpu.VMEM((_RJ, _RL), jnp.int32),
            pltpu.VMEM((_RJ, _RL), jnp.float32),
            pltpu.VMEM((8, 128), jnp.float32),
            pltpu.VMEM((8 * _NS, 128), jnp.float32),
            pltpu.VMEM((_LANES,), jnp.float32),
            pltpu.VMEM_SHARED((8 * _NS, 128), jnp.float32),
            pltpu.SemaphoreType.DMA,
        ],
    )
    def body(preds_hbm, tgt_hbm, mar_hbm, out_hbm,
             tgt_v, mar_v, w_v, idx_v, val_v, blk_v, mat_v, row_v,
             shared, sem):
        c = lax.axis_index("c")
        s = lax.axis_index("s")
        wid = s * _NC + c
        base = wid * _PER

        pltpu.sync_copy(tgt_hbm.at[pl.ds(base, _PER)], tgt_v)

        lane = lax.iota(jnp.int32, _LANES)
        for j in range(_PER // _LANES):
            t = tgt_v[pl.ds(j * _LANES, _LANES)]
            idx = (
                (t >> 3) * (_B * 8)
                + (wid * 4 + j // 8) * 1024
                + (t & 7) * 128
                + (j % 8) * _LANES
                + lane
            )
            idx_v[j // 8, pl.ds((j % 8) * _LANES, _LANES)] = idx

        copies = [
            pltpu.async_copy(preds_hbm.at[idx_v.at[j]], val_v.at[j], sem)
            for j in range(_RJ)
        ]

        pltpu.sync_copy(mar_hbm.at[pl.ds(base, _PER)], mar_v)
        for j in range(_PER // _LANES):
            m = mar_v[pl.ds(j * _LANES, _LANES)]
            w = jnp.exp(-0.5 * m * m)
            w_v[pl.ds(j * _LANES, _LANES)] = jnp.where(
                m != 0.0, w, jnp.zeros_like(w)
            )

        for cp in copies:
            cp.wait()

        acc = jnp.zeros((_LANES,), jnp.float32)
        for j in range(_PER // _LANES):
            v = val_v[j // 8, pl.ds((j % 8) * _LANES, _LANES)]
            w = w_v[pl.ds(j * _LANES, _LANES)]
            acc = acc + w * v

        blk_v[0, pl.ds(0, _LANES)] = acc
        pltpu.sync_copy(blk_v, shared.at[pl.ds(s * 8, 8), pl.ds(0, 128)])
        plsc.subcore_barrier()

        @pl.when(s == 0)
        def _():
            pltpu.sync_copy(shared, mat_v)
            v = mat_v[0, pl.ds(0, _LANES)]
            for k in range(1, _NS):
                v = v + mat_v[k * 8, pl.ds(0, _LANES)]
            for shift in (8, 4, 2, 1):
                row_v[...] = v
                v = v + plsc.load_gather(row_v, [(lane + shift) & (_LANES - 1)])
            row_v[...] = v * (-1.0 / _B)
            pltpu.sync_copy(row_v, out_hbm.at[c])

    return body(preds_flat, targets, margin)


def kernel(preds, targets, margin):
    preds_flat = (
        preds.T.reshape(_C // 8, 8, _B // 128, 128)
        .transpose(0, 2, 1, 3)
        .reshape(_B * _C)
    )
    core_sums = _sc_loss(preds_flat, targets.astype(jnp.int32), margin)
    return core_sums[0, 0] + core_sums[1, 0]


# speedup vs baseline: 1.1066x; 1.1066x over previous
import functools

import jax
import jax.numpy as jnp
from jax import lax
from jax.experimental import pallas as pl
from jax.experimental.pallas import tpu as pltpu
from jax.experimental.pallas import tpu_sc as plsc

_B = 16384
_C = 1000
_NC, _NS = 2, 16
_NW = _NC * _NS
_PER = _B // _NW
_RJ = 4
_RL = _PER // _RJ
_LANES = 16


def _sc_partials(preds_flat, targets, margin):
    mesh = plsc.VectorSubcoreMesh(core_axis_name="c", subcore_axis_name="s")

    @functools.partial(
        pl.kernel,
        mesh=mesh,
        out_type=jax.ShapeDtypeStruct((_NW, _LANES), jnp.float32),
        compiler_params=pltpu.CompilerParams(needs_layout_passes=False),
        scratch_types=[
            pltpu.VMEM((_PER,), jnp.int32),
            pltpu.VMEM((_PER,), jnp.float32),
            pltpu.VMEM((_PER,), jnp.float32),
            pltpu.VMEM((_RJ, _RL), jnp.int32),
            pltpu.VMEM((_RJ, _RL), jnp.float32),
            pltpu.VMEM((_LANES,), jnp.float32),
            pltpu.SemaphoreType.DMA,
        ],
    )
    def body(preds_hbm, tgt_hbm, mar_hbm, out_hbm,
             tgt_v, mar_v, w_v, idx_v, val_v, row_v, sem):
        c = lax.axis_index("c")
        s = lax.axis_index("s")
        wid = s * _NC + c
        base = wid * _PER

        pltpu.sync_copy(tgt_hbm.at[pl.ds(base, _PER)], tgt_v)

        lane = lax.iota(jnp.int32, _LANES)
        copies = []
        for jj in range(_RJ):
            for i in range(_RL // _LANES):
                j = jj * (_RL // _LANES) + i
                t = tgt_v[pl.ds(j * _LANES, _LANES)]
                idx = (
                    (t >> 3) * (_B * 8)
                    + (wid * 4 + jj) * 1024
                    + i * _LANES
                    + (t & 7) * 128
                    + lane
                )
                idx_v[jj, pl.ds(i * _LANES, _LANES)] = idx
            copies.append(
                pltpu.async_copy(preds_hbm.at[idx_v.at[jj]], val_v.at[jj], sem)
            )

        pltpu.sync_copy(mar_hbm.at[pl.ds(base, _PER)], mar_v)
        for j in range(_PER // _LANES):
            m = mar_v[pl.ds(j * _LANES, _LANES)]
            w = jnp.exp(-0.5 * m * m)
            w_v[pl.ds(j * _LANES, _LANES)] = jnp.where(
                m != 0.0, w, jnp.zeros_like(w)
            )

        for cp in copies:
            cp.wait()

        acc = jnp.zeros((_LANES,), jnp.float32)
        for j in range(_PER // _LANES):
            v = val_v[j // 8, pl.ds((j % 8) * _LANES, _LANES)]
            w = w_v[pl.ds(j * _LANES, _LANES)]
            acc = acc + w * v
        row_v[...] = acc
        pltpu.sync_copy(row_v, out_hbm.at[wid])

    return body(preds_flat, targets, margin)


def kernel(preds, targets, margin):
    preds_flat = (
        preds.T.reshape(_C // 8, 8, _B // 128, 128)
        .transpose(0, 2, 1, 3)
        .reshape(_B * _C)
    )
    partials = _sc_partials(preds_flat, targets.astype(jnp.int32), margin)

    def tc_body(x_ref, o_ref):
        total = jnp.sum(x_ref[...]) * (-1.0 / _B)
        o_ref[...] = jnp.broadcast_to(total, (1, 1))

    loss = pl.pallas_call(
        tc_body,
        out_shape=jax.ShapeDtypeStruct((1, 1), jnp.float32),
    )(partials)
    return loss[0, 0]


# speedup vs baseline: 1.1067x; 1.0001x over previous
import functools

import jax
import jax.numpy as jnp
from jax import lax
from jax.experimental import pallas as pl
from jax.experimental.pallas import tpu as pltpu
from jax.experimental.pallas import tpu_sc as plsc

_B = 16384
_C = 1000
_NC, _NS = 2, 16
_NW = _NC * _NS
_PER = _B // _NW
_RJ = 4
_RL = _PER // _RJ
_LANES = 16


def _sc_partials(preds_flat, targets, margin):
    mesh = plsc.VectorSubcoreMesh(core_axis_name="c", subcore_axis_name="s")

    @functools.partial(
        pl.kernel,
        mesh=mesh,
        out_type=jax.ShapeDtypeStruct((_NW, _LANES), jnp.float32),
        compiler_params=pltpu.CompilerParams(needs_layout_passes=False),
        scratch_types=[
            pltpu.VMEM((_PER,), jnp.int32),
            pltpu.VMEM((_PER,), jnp.float32),
            pltpu.VMEM((_PER,), jnp.float32),
            pltpu.VMEM((_RJ, _RL), jnp.int32),
            pltpu.VMEM((_RJ, _RL), jnp.float32),
            pltpu.VMEM((_LANES,), jnp.float32),
            pltpu.SemaphoreType.DMA,
        ],
    )
    def body(preds_hbm, tgt_hbm, mar_hbm, out_hbm,
             tgt_v, mar_v, w_v, idx_v, val_v, row_v, sem):
        c = lax.axis_index("c")
        s = lax.axis_index("s")
        wid = s * _NC + c
        base = wid * _PER

        pltpu.sync_copy(tgt_hbm.at[pl.ds(base, _PER)], tgt_v)

        lane = lax.iota(jnp.int32, _LANES)
        copies = []
        for jj in range(_RJ):
            for i in range(_RL // _LANES):
                j = jj * (_RL // _LANES) + i
                t = tgt_v[pl.ds(j * _LANES, _LANES)]
                idx = (
                    (t >> 3) * (_B * 8)
                    + (wid * 4 + jj) * 1024
                    + i * _LANES
                    + (t & 7) * 128
                    + lane
                )
                idx_v[jj, pl.ds(i * _LANES, _LANES)] = idx
            copies.append(
                pltpu.async_copy(preds_hbm.at[idx_v.at[jj]], val_v.at[jj], sem)
            )

        pltpu.sync_copy(mar_hbm.at[pl.ds(base, _PER)], mar_v)
        for j in range(_PER // _LANES):
            m = mar_v[pl.ds(j * _LANES, _LANES)]
            w = jnp.exp(-0.5 * m * m)
            w_v[pl.ds(j * _LANES, _LANES)] = jnp.where(
                m != 0.0, w, jnp.zeros_like(w)
            )

        for cp in copies:
            cp.wait()

        acc = jnp.zeros((_LANES,), jnp.float32)
        for j in range(_PER // _LANES):
            v = val_v[j // 8, pl.ds((j % 8) * _LANES, _LANES)]
            w = w_v[pl.ds(j * _LANES, _LANES)]
            acc = acc + w * v
        row_v[...] = acc
        pltpu.sync_copy(row_v, out_hbm.at[wid])

    return body(preds_flat, targets, margin)


def kernel(preds, targets, margin):
    preds_flat = (
        preds.T.reshape(_C // 8, 8, _B // 128, 128)
        .transpose(0, 2, 1, 3)
        .reshape(_B * _C)
    )
    partials = _sc_partials(preds_flat, targets.astype(jnp.int32), margin)

    def tc_body(x_hbm, o_ref, x_vmem, sem):
        cp = pltpu.make_async_copy(x_hbm, x_vmem, sem)
        cp.start()
        cp.wait()
        total = jnp.sum(x_vmem[...]) * (-1.0 / _B)
        o_ref[...] = jnp.broadcast_to(total, (1, 1))

    loss = pl.pallas_call(
        tc_body,
        in_specs=[pl.BlockSpec(memory_space=pltpu.MemorySpace.HBM)],
        out_shape=jax.ShapeDtypeStruct((1, 1), jnp.float32),
        scratch_shapes=[
            pltpu.VMEM((_NW, _LANES), jnp.float32),
            pltpu.SemaphoreType.DMA,
        ],
    )(partials)
    return loss[0, 0]


# speedup vs baseline: 1.1127x; 1.0054x over previous
import functools

import jax
import jax.numpy as jnp
from jax import lax
from jax.experimental import pallas as pl
from jax.experimental.pallas import tpu as pltpu
from jax.experimental.pallas import tpu_sc as plsc

_B = 16384
_C = 1000
_NC, _NS = 2, 16
_NW = _NC * _NS
_PER = _B // _NW
_RJ = 8
_RL = _PER // _RJ
_LANES = 16


def _sc_partials(preds_flat, targets, margin):
    mesh = plsc.VectorSubcoreMesh(core_axis_name="c", subcore_axis_name="s")

    @functools.partial(
        pl.kernel,
        mesh=mesh,
        out_type=jax.ShapeDtypeStruct((_NW, _LANES), jnp.float32),
        compiler_params=pltpu.CompilerParams(needs_layout_passes=False),
        scratch_types=[
            pltpu.VMEM((_PER,), jnp.int32),
            pltpu.VMEM((_PER,), jnp.float32),
            pltpu.VMEM((_PER,), jnp.float32),
            pltpu.VMEM((_RJ, _RL), jnp.int32),
            pltpu.VMEM((_RJ, _RL), jnp.float32),
            pltpu.VMEM((_LANES,), jnp.float32),
            pltpu.SemaphoreType.DMA,
        ],
    )
    def body(preds_hbm, tgt_hbm, mar_hbm, out_hbm,
             tgt_v, mar_v, w_v, idx_v, val_v, row_v, sem):
        c = lax.axis_index("c")
        s = lax.axis_index("s")
        wid = s * _NC + c
        base = wid * _PER

        pltpu.sync_copy(tgt_hbm.at[pl.ds(base, _PER)], tgt_v)

        lane = lax.iota(jnp.int32, _LANES)
        copies = []
        for jj in range(_RJ):
            for i in range(_RL // _LANES):
                l = jj * _RL + i * _LANES
                t = tgt_v[pl.ds(l, _LANES)]
                idx = (
                    (t >> 3) * (_B * 8)
                    + (wid * 4 + l // 128) * 1024
                    + (l % 128)
                    + (t & 7) * 128
                    + lane
                )
                idx_v[jj, pl.ds(i * _LANES, _LANES)] = idx
            copies.append(
                pltpu.async_copy(preds_hbm.at[idx_v.at[jj]], val_v.at[jj], sem)
            )

        pltpu.sync_copy(mar_hbm.at[pl.ds(base, _PER)], mar_v)
        for j in range(_PER // _LANES):
            m = mar_v[pl.ds(j * _LANES, _LANES)]
            w = jnp.exp(-0.5 * m * m)
            w_v[pl.ds(j * _LANES, _LANES)] = jnp.where(
                m != 0.0, w, jnp.zeros_like(w)
            )

        for cp in copies:
            cp.wait()

        acc = jnp.zeros((_LANES,), jnp.float32)
        for j in range(_PER // _LANES):
            g = _RL // _LANES
            v = val_v[j // g, pl.ds((j % g) * _LANES, _LANES)]
            w = w_v[pl.ds(j * _LANES, _LANES)]
            acc = acc + w * v
        row_v[...] = acc
        pltpu.sync_copy(row_v, out_hbm.at[wid])

    return body(preds_flat, targets, margin)


def kernel(preds, targets, margin):
    preds_flat = (
        preds.T.reshape(_C // 8, 8, _B // 128, 128)
        .transpose(0, 2, 1, 3)
        .reshape(_B * _C)
    )
    partials = _sc_partials(preds_flat, targets.astype(jnp.int32), margin)

    def tc_body(x_hbm, o_ref, x_vmem, sem):
        cp = pltpu.make_async_copy(x_hbm, x_vmem, sem)
        cp.start()
        cp.wait()
        total = jnp.sum(x_vmem[...]) * (-1.0 / _B)
        o_ref[...] = jnp.broadcast_to(total, (1, 1))

    loss = pl.pallas_call(
        tc_body,
        in_specs=[pl.BlockSpec(memory_space=pltpu.MemorySpace.HBM)],
        out_shape=jax.ShapeDtypeStruct((1, 1), jnp.float32),
        scratch_shapes=[
            pltpu.VMEM((_NW, _LANES), jnp.float32),
            pltpu.SemaphoreType.DMA,
        ],
    )(partials)
    return loss[0, 0]


# speedup vs baseline: 1.1154x; 1.0024x over previous
import functools

import jax
import jax.numpy as jnp
from jax import lax
from jax.experimental import pallas as pl
from jax.experimental.pallas import tpu as pltpu
from jax.experimental.pallas import tpu_sc as plsc

_B = 16384
_C = 1000
_NC, _NS = 1, 16
_NW = _NC * _NS
_PER = _B // _NW
_RJ = 8
_RL = _PER // _RJ
_LANES = 16


def _sc_partials(preds_flat, targets, margin):
    mesh = plsc.VectorSubcoreMesh(
        core_axis_name="c", subcore_axis_name="s", num_cores=_NC
    )

    @functools.partial(
        pl.kernel,
        mesh=mesh,
        out_type=jax.ShapeDtypeStruct((_NW, _LANES), jnp.float32),
        compiler_params=pltpu.CompilerParams(needs_layout_passes=False),
        scratch_types=[
            pltpu.VMEM((_PER,), jnp.int32),
            pltpu.VMEM((_PER,), jnp.float32),
            pltpu.VMEM((_PER,), jnp.float32),
            pltpu.VMEM((_RJ, _RL), jnp.int32),
            pltpu.VMEM((_RJ, _RL), jnp.float32),
            pltpu.VMEM((_LANES,), jnp.float32),
            pltpu.SemaphoreType.DMA,
        ],
    )
    def body(preds_hbm, tgt_hbm, mar_hbm, out_hbm,
             tgt_v, mar_v, w_v, idx_v, val_v, row_v, sem):
        c = lax.axis_index("c")
        s = lax.axis_index("s")
        wid = s * _NC + c
        base = wid * _PER

        pltpu.sync_copy(tgt_hbm.at[pl.ds(base, _PER)], tgt_v)

        lane = lax.iota(jnp.int32, _LANES)
        copies = []
        for jj in range(_RJ):
            for i in range(_RL // _LANES):
                l = jj * _RL + i * _LANES
                t = tgt_v[pl.ds(l, _LANES)]
                idx = (
                    (t >> 3) * (_B * 8)
                    + (wid * (_PER // 128) + l // 128) * 1024
                    + (l % 128)
                    + (t & 7) * 128
                    + lane
                )
                idx_v[jj, pl.ds(i * _LANES, _LANES)] = idx
            copies.append(
                pltpu.async_copy(preds_hbm.at[idx_v.at[jj]], val_v.at[jj], sem)
            )

        pltpu.sync_copy(mar_hbm.at[pl.ds(base, _PER)], mar_v)
        for j in range(_PER // _LANES):
            m = mar_v[pl.ds(j * _LANES, _LANES)]
            w = jnp.exp(-0.5 * m * m)
            w_v[pl.ds(j * _LANES, _LANES)] = jnp.where(
                m != 0.0, w, jnp.zeros_like(w)
            )

        for cp in copies:
            cp.wait()

        acc = jnp.zeros((_LANES,), jnp.float32)
        for j in range(_PER // _LANES):
            g = _RL // _LANES
            v = val_v[j // g, pl.ds((j % g) * _LANES, _LANES)]
            w = w_v[pl.ds(j * _LANES, _LANES)]
            acc = acc + w * v
        row_v[...] = acc
        pltpu.sync_copy(row_v, out_hbm.at[wid])

    return body(preds_flat, targets, margin)


def kernel(preds, targets, margin):
    preds_flat = (
        preds.T.reshape(_C // 8, 8, _B // 128, 128)
        .transpose(0, 2, 1, 3)
        .reshape(_B * _C)
    )
    partials = _sc_partials(preds_flat, targets.astype(jnp.int32), margin)

    def tc_body(x_hbm, o_ref, x_vmem, sem):
        cp = pltpu.make_async_copy(x_hbm, x_vmem, sem)
        cp.start()
        cp.wait()
        total = jnp.sum(x_vmem[...]) * (-1.0 / _B)
        o_ref[...] = jnp.broadcast_to(total, (1, 1))

    loss = pl.pallas_call(
        tc_body,
        in_specs=[pl.BlockSpec(memory_space=pltpu.MemorySpace.HBM)],
        out_shape=jax.ShapeDtypeStruct((1, 1), jnp.float32),
        scratch_shapes=[
            pltpu.VMEM((_NW, _LANES), jnp.float32),
            pltpu.SemaphoreType.DMA,
        ],
    )(partials)
    return loss[0, 0]
